# baseline (speedup 1.0000x reference)
import functools

import jax
import jax.numpy as jnp
from jax import lax
from jax.experimental import pallas as pl
from jax.experimental.pallas import tpu as pltpu
from jax.experimental.pallas import tpu_sc as plsc

SEQ = 4096
D = 1024
NC = 2
NS = 16
NW = NC * NS
ROWS_PER_W = SEQ // NW
CH = 32
NCHUNK = ROWS_PER_W // CH

_mesh = plsc.VectorSubcoreMesh(core_axis_name="c", subcore_axis_name="s")


NBUF = 3


@functools.partial(
    pl.kernel,
    mesh=_mesh,
    out_type=jax.ShapeDtypeStruct((1, SEQ, D), jnp.float32),
    scratch_types=(
        [pltpu.VMEM((CH, D), jnp.float32)] * NBUF
        + [pltpu.SemaphoreType.DMA] * (2 * NBUF)
    ),
)
def _slice_copy(enc_hbm, out_hbm, *scratch):
    bufs = scratch[:NBUF]
    in_sems = scratch[NBUF : 2 * NBUF]
    out_sems = scratch[2 * NBUF :]
    wid = lax.axis_index("s") * NC + lax.axis_index("c")
    base = wid * ROWS_PER_W

    in_copies = [None] * NCHUNK
    out_copies = [None] * NCHUNK
    for i in range(NCHUNK):
        b = i % NBUF
        if i >= NBUF:
            out_copies[i - NBUF].wait()
        in_copies[i] = pltpu.async_copy(
            enc_hbm.at[pl.ds(base + i * CH, CH), :], bufs[b], in_sems[b]
        )
        if i >= 1:
            in_copies[i - 1].wait()
            out_copies[i - 1] = pltpu.async_copy(
                bufs[(i - 1) % NBUF],
                out_hbm.at[0, pl.ds(base + (i - 1) * CH, CH), :],
                out_sems[(i - 1) % NBUF],
            )
    in_copies[NCHUNK - 1].wait()
    out_copies[NCHUNK - 1] = pltpu.async_copy(
        bufs[(NCHUNK - 1) % NBUF],
        out_hbm.at[0, pl.ds(base + (NCHUNK - 1) * CH, CH), :],
        out_sems[(NCHUNK - 1) % NBUF],
    )
    for i in range(max(0, NCHUNK - NBUF), NCHUNK):
        out_copies[i].wait()


def kernel(x, encoding):
    del x
    return _slice_copy(encoding)


# speedup vs baseline: 1.0133x; 1.0133x over previous
import functools

import jax
import jax.numpy as jnp
from jax import lax
from jax.experimental import pallas as pl
from jax.experimental.pallas import tpu as pltpu
from jax.experimental.pallas import tpu_sc as plsc

SEQ = 4096
D = 1024
NC = 2
NS = 16
NW = NC * NS
ROWS_PER_W = SEQ // NW
CHUNKS = (8, 24, 32, 32, 32)
CHMAX = max(CHUNKS)
NCHUNK = len(CHUNKS)
OFFS = [sum(CHUNKS[:i]) for i in range(NCHUNK)]
NBUF = 3

_mesh = plsc.VectorSubcoreMesh(core_axis_name="c", subcore_axis_name="s")


@functools.partial(
    pl.kernel,
    mesh=_mesh,
    out_type=jax.ShapeDtypeStruct((1, SEQ, D), jnp.float32),
    scratch_types=(
        [pltpu.VMEM((CHMAX, D), jnp.float32)] * NBUF
        + [pltpu.SemaphoreType.DMA] * (2 * NBUF)
    ),
)
def _slice_copy(enc_hbm, out_hbm, *scratch):
    bufs = scratch[:NBUF]
    in_sems = scratch[NBUF : 2 * NBUF]
    out_sems = scratch[2 * NBUF :]
    wid = lax.axis_index("s") * NC + lax.axis_index("c")
    base = wid * ROWS_PER_W

    in_copies = [None] * NCHUNK
    out_copies = [None] * NCHUNK

    def _scatter(i):
        b = i % NBUF
        in_copies[i].wait()
        out_copies[i] = pltpu.async_copy(
            bufs[b].at[pl.ds(0, CHUNKS[i]), :],
            out_hbm.at[0, pl.ds(base + OFFS[i], CHUNKS[i]), :],
            out_sems[b],
        )

    for i in range(NCHUNK):
        b = i % NBUF
        if i >= NBUF:
            out_copies[i - NBUF].wait()
        in_copies[i] = pltpu.async_copy(
            enc_hbm.at[pl.ds(base + OFFS[i], CHUNKS[i]), :],
            bufs[b].at[pl.ds(0, CHUNKS[i]), :],
            in_sems[b],
        )
        if i >= 1:
            _scatter(i - 1)
    _scatter(NCHUNK - 1)
    for i in range(max(0, NCHUNK - NBUF), NCHUNK):
        out_copies[i].wait()


def kernel(x, encoding):
    del x
    return _slice_copy(encoding)
